# Initial kernel scaffold; baseline (speedup 1.0000x reference)
import jax
import jax.numpy as jnp
from jax.experimental import pallas as pl


def kernel(query, W_off, b_off, W_attn, b_attn, W_val, b_val, W_out, b_out):
    raise NotImplementedError("write your pallas kernel here")


# speedup vs baseline: 6711.6942x; 6711.6942x over previous
import functools

import jax
import jax.numpy as jnp
import numpy as np
from jax import lax
from jax.experimental import pallas as pl
from jax.experimental.pallas import tpu as pltpu
from jax.experimental.pallas import tpu_sc as plsc

NUM_H = 64
EMBED = 256
HEADS = 8
POINTS = 4
HEAD_DIM = EMBED // HEADS
BS = 4
NQ = NUM_H * NUM_H
NBQ = BS * NQ
NROWS = NBQ * HEADS
NS = 4 * POINTS * HEADS

QB = 512
GRID = NBQ // QB

SC_CORES = 2
SC_SUBCORES = 16
NW = SC_CORES * SC_SUBCORES
QPW = NBQ // NW
TQ = 16
NCH = QPW // TQ
SAMP = TQ * NS


def _prep_body(q_ref, wval_ref, bval_ref, woffx_ref, boffx_ref, woffy_ref,
               boffy_ref, wattn_ref, battn_ref, val_ref, idx_ref, w_ref):
    q = q_ref[:]
    f32 = jnp.float32
    val_ref[:] = jnp.dot(q, wval_ref[:], preferred_element_type=f32) + bval_ref[:]
    offx = jnp.dot(q, woffx_ref[:], preferred_element_type=f32) + boffx_ref[:]
    offy = jnp.dot(q, woffy_ref[:], preferred_element_type=f32) + boffy_ref[:]
    logits = jnp.dot(q, wattn_ref[:], preferred_element_type=f32) + battn_ref[:]

    e = jnp.exp(logits)
    gi = lax.broadcasted_iota(jnp.int32, (32, 32), 0) >> 2
    gj = lax.broadcasted_iota(jnp.int32, (32, 32), 1) >> 2
    G = (gi == gj).astype(f32)
    attnw = e / jnp.dot(e, G, preferred_element_type=f32)

    rowid = pl.program_id(0) * QB + lax.broadcasted_iota(jnp.int32, (QB, 1), 0)
    b = rowid >> 12
    rc = rowid & (NQ - 1)
    r = rc >> 6
    c = rc & (NUM_H - 1)
    scale = np.float32(NUM_H / (NUM_H - 1.0))
    ix = c.astype(f32) * scale - 0.5 + offx
    iy = r.astype(f32) * scale - 0.5 + offy

    x0f = jnp.floor(ix)
    y0f = jnp.floor(iy)
    fx = ix - x0f
    fy = iy - y0f
    x0 = jnp.clip(x0f, -2.0, 65.0).astype(jnp.int32)
    y0 = jnp.clip(y0f, -2.0, 65.0).astype(jnp.int32)
    x1 = x0 + 1
    y1 = y0 + 1

    def v(t):
        return ((t >= 0) & (t <= NUM_H - 1)).astype(f32)

    vx0, vx1, vy0, vy1 = v(x0), v(x1), v(y0), v(y1)
    xc0 = jnp.clip(x0, 0, NUM_H - 1)
    xc1 = jnp.clip(x1, 0, NUM_H - 1)
    yc0 = jnp.clip(y0, 0, NUM_H - 1)
    yc1 = jnp.clip(y1, 0, NUM_H - 1)

    h_lane = lax.broadcasted_iota(jnp.int32, (1, 32), 1) >> 2
    sb = b << 12

    def rid(yc, xc):
        return ((sb + (yc << 6) + xc) << 3) + h_lane

    idx_ref[:, 0:32] = rid(yc0, xc0)
    idx_ref[:, 32:64] = rid(yc1, xc0)
    idx_ref[:, 64:96] = rid(yc0, xc1)
    idx_ref[:, 96:128] = rid(yc1, xc1)

    wx0 = 1.0 - fx
    wy0 = 1.0 - fy
    w_ref[:, 0:32] = attnw * wx0 * wy0 * vx0 * vy0
    w_ref[:, 32:64] = attnw * wx0 * fy * vx0 * vy1
    w_ref[:, 64:96] = attnw * fx * wy0 * vx1 * vy0
    w_ref[:, 96:128] = attnw * fx * fy * vx1 * vy1


def _prep_call(qf, W_val, b_val, woffx, boffx, woffy, boffy, W_attn, b_attn,
               interpret=False):
    full = lambda s: pl.BlockSpec(s, lambda i: (0, 0))
    return pl.pallas_call(
        _prep_body,
        grid=(GRID,),
        in_specs=[
            pl.BlockSpec((QB, EMBED), lambda i: (i, 0)),
            full((EMBED, EMBED)), full((1, EMBED)),
            full((EMBED, 32)), full((1, 32)),
            full((EMBED, 32)), full((1, 32)),
            full((EMBED, 32)), full((1, 32)),
        ],
        out_specs=[
            pl.BlockSpec((QB, EMBED), lambda i: (i, 0)),
            pl.BlockSpec((QB, NS), lambda i: (i, 0)),
            pl.BlockSpec((QB, NS), lambda i: (i, 0)),
        ],
        out_shape=[
            jax.ShapeDtypeStruct((NBQ, EMBED), jnp.float32),
            jax.ShapeDtypeStruct((NBQ, NS), jnp.int32),
            jax.ShapeDtypeStruct((NBQ, NS), jnp.float32),
        ],
        interpret=interpret,
    )(qf, W_val, b_val, woffx, boffx, woffy, boffy, W_attn, b_attn)


def _sc_body(table_hbm, idx_hbm, w_hbm, out_hbm, idx_v, w_v, rows_v, out_v, sem):
    wid = lax.axis_index("s") * SC_CORES + lax.axis_index("c")

    def chunk_body(ch, carry):
        q0 = wid * QPW + ch * TQ
        pltpu.sync_copy(idx_hbm.at[pl.ds(q0, TQ)], idx_v)
        pltpu.sync_copy(w_hbm.at[pl.ds(q0 * NS, SAMP)], w_v)
        copies = [
            pltpu.async_copy(table_hbm.at[idx_v.at[k]],
                             rows_v.at[pl.ds(k * NS, NS)], sem)
            for k in range(TQ)
        ]
        for cp in copies:
            cp.wait()

        def q_body(qq, carry2):
            base = qq * NS
            for h in range(HEADS):
                acc0 = jnp.zeros((16,), jnp.float32)
                acc1 = jnp.zeros((16,), jnp.float32)
                for c4 in range(4):
                    for p in range(POINTS):
                        pos = base + c4 * 32 + h * 4 + p
                        wspl = plsc.load_gather(
                            w_v, [jnp.broadcast_to(pos, (16,)).astype(jnp.int32)])
                        acc0 = acc0 + wspl * rows_v[pos, pl.ds(0, 16)]
                        acc1 = acc1 + wspl * rows_v[pos, pl.ds(16, 16)]
                out_v[qq * HEADS + h, pl.ds(0, 16)] = acc0
                out_v[qq * HEADS + h, pl.ds(16, 16)] = acc1
            return carry2

        lax.fori_loop(0, TQ, q_body, 0)
        pltpu.sync_copy(out_v, out_hbm.at[pl.ds(q0 * HEADS, TQ * HEADS)])
        return carry

    lax.fori_loop(0, NCH, chunk_body, 0)


def _sc_call(table, idx, w_flat):
    mesh = plsc.VectorSubcoreMesh(core_axis_name="c", subcore_axis_name="s")
    return pl.kernel(
        _sc_body,
        out_type=jax.ShapeDtypeStruct((NROWS, HEAD_DIM), jnp.float32),
        mesh=mesh,
        scratch_types=[
            pltpu.VMEM((TQ, NS), jnp.int32),
            pltpu.VMEM((SAMP,), jnp.float32),
            pltpu.VMEM((SAMP, HEAD_DIM), jnp.float32),
            pltpu.VMEM((TQ * HEADS, HEAD_DIM), jnp.float32),
            pltpu.SemaphoreType.DMA,
        ],
        compiler_params=pltpu.CompilerParams(needs_layout_passes=False,
                                             use_tc_tiling_on_sc=False),
    )(table, idx, w_flat)


def _out_body(s_ref, q_ref, wout_ref, bout_ref, o_ref):
    o_ref[:] = (jnp.dot(s_ref[:], wout_ref[:], preferred_element_type=jnp.float32)
                + bout_ref[:] + 2.0 * q_ref[:])


def _out_call(smp, qf, W_out, b_out, interpret=False):
    return pl.pallas_call(
        _out_body,
        grid=(GRID,),
        in_specs=[
            pl.BlockSpec((QB, EMBED), lambda i: (i, 0)),
            pl.BlockSpec((QB, EMBED), lambda i: (i, 0)),
            pl.BlockSpec((EMBED, EMBED), lambda i: (0, 0)),
            pl.BlockSpec((1, EMBED), lambda i: (0, 0)),
        ],
        out_specs=pl.BlockSpec((QB, EMBED), lambda i: (i, 0)),
        out_shape=jax.ShapeDtypeStruct((NBQ, EMBED), jnp.float32),
        interpret=interpret,
    )(smp, qf, W_out, b_out)


def kernel(query, W_off, b_off, W_attn, b_attn, W_val, b_val, W_out, b_out):
    qf = query.reshape(NBQ, EMBED)
    woffx = W_off[:, 0::2]
    woffy = W_off[:, 1::2]
    boffx = b_off[0::2].reshape(1, 32)
    boffy = b_off[1::2].reshape(1, 32)

    val, idx, w = _prep_call(qf, W_val, b_val.reshape(1, EMBED), woffx, boffx,
                             woffy, boffy, W_attn, b_attn.reshape(1, 32))
    table = val.reshape(NROWS, HEAD_DIM)
    smp = _sc_call(table, idx, w.reshape(-1))
    out = _out_call(smp.reshape(NBQ, EMBED), qf, W_out, b_out.reshape(1, EMBED))
    return out.reshape(BS, NQ, EMBED)


# speedup vs baseline: 10106.5687x; 1.5058x over previous
import functools

import jax
import jax.numpy as jnp
import numpy as np
from jax import lax
from jax.experimental import pallas as pl
from jax.experimental.pallas import tpu as pltpu
from jax.experimental.pallas import tpu_sc as plsc

NUM_H = 64
EMBED = 256
HEADS = 8
POINTS = 4
HEAD_DIM = EMBED // HEADS
BS = 4
NQ = NUM_H * NUM_H
NBQ = BS * NQ
NROWS = NBQ * HEADS
NS = 4 * POINTS * HEADS

QB = 512
GRID = NBQ // QB

SC_CORES = 2
SC_SUBCORES = 16
NW = SC_CORES * SC_SUBCORES
QPW = NBQ // NW
TQ = 16
NCH = QPW // TQ
SAMP = TQ * NS


def _prep_body(q_ref, wval_ref, bval_ref, woffx_ref, boffx_ref, woffy_ref,
               boffy_ref, wattn_ref, battn_ref, val_ref, pk_ref):
    q = q_ref[:]
    f32 = jnp.float32
    val = jnp.dot(q, wval_ref[:], preferred_element_type=f32) + bval_ref[:]
    val_ref[:] = val.astype(jnp.bfloat16)
    offx = jnp.dot(q, woffx_ref[:], preferred_element_type=f32) + boffx_ref[:]
    offy = jnp.dot(q, woffy_ref[:], preferred_element_type=f32) + boffy_ref[:]
    logits = jnp.dot(q, wattn_ref[:], preferred_element_type=f32) + battn_ref[:]

    e = jnp.exp(logits)
    gi = lax.broadcasted_iota(jnp.int32, (32, 32), 0) >> 2
    gj = lax.broadcasted_iota(jnp.int32, (32, 32), 1) >> 2
    G = (gi == gj).astype(f32)
    attnw = e / jnp.dot(e, G, preferred_element_type=f32)

    rowid = pl.program_id(0) * QB + lax.broadcasted_iota(jnp.int32, (QB, 1), 0)
    b = rowid >> 12
    rc = rowid & (NQ - 1)
    r = rc >> 6
    c = rc & (NUM_H - 1)
    scale = np.float32(NUM_H / (NUM_H - 1.0))
    ix = c.astype(f32) * scale - 0.5 + offx
    iy = r.astype(f32) * scale - 0.5 + offy

    x0f = jnp.floor(ix)
    y0f = jnp.floor(iy)
    fx = ix - x0f
    fy = iy - y0f
    x0 = jnp.clip(x0f, -2.0, 65.0).astype(jnp.int32)
    y0 = jnp.clip(y0f, -2.0, 65.0).astype(jnp.int32)
    x1 = x0 + 1
    y1 = y0 + 1

    def v(t):
        return ((t >= 0) & (t <= NUM_H - 1)).astype(f32)

    vx0, vx1, vy0, vy1 = v(x0), v(x1), v(y0), v(y1)
    xc0 = jnp.clip(x0, 0, NUM_H - 1)
    xc1 = jnp.clip(x1, 0, NUM_H - 1)
    yc0 = jnp.clip(y0, 0, NUM_H - 1)
    yc1 = jnp.clip(y1, 0, NUM_H - 1)

    h_lane = lax.broadcasted_iota(jnp.int32, (1, 32), 1) >> 2
    sb = b << 12

    def rid(yc, xc):
        return ((sb + (yc << 6) + xc) << 3) + h_lane

    pk_ref[:, 0:32] = rid(yc0, xc0)
    pk_ref[:, 32:64] = rid(yc1, xc0)
    pk_ref[:, 64:96] = rid(yc0, xc1)
    pk_ref[:, 96:128] = rid(yc1, xc1)

    wx0 = 1.0 - fx
    wy0 = 1.0 - fy

    def wbits(w):
        return lax.bitcast_convert_type(w, jnp.int32)

    pk_ref[:, 128:160] = wbits(attnw * wx0 * wy0 * vx0 * vy0)
    pk_ref[:, 160:192] = wbits(attnw * wx0 * fy * vx0 * vy1)
    pk_ref[:, 192:224] = wbits(attnw * fx * wy0 * vx1 * vy0)
    pk_ref[:, 224:256] = wbits(attnw * fx * fy * vx1 * vy1)


def _prep_call(qf, W_val, b_val, woffx, boffx, woffy, boffy, W_attn, b_attn,
               interpret=False):
    full = lambda s: pl.BlockSpec(s, lambda i: (0, 0))
    return pl.pallas_call(
        _prep_body,
        grid=(GRID,),
        in_specs=[
            pl.BlockSpec((QB, EMBED), lambda i: (i, 0)),
            full((EMBED, EMBED)), full((1, EMBED)),
            full((EMBED, 32)), full((1, 32)),
            full((EMBED, 32)), full((1, 32)),
            full((EMBED, 32)), full((1, 32)),
        ],
        out_specs=[
            pl.BlockSpec((QB, EMBED), lambda i: (i, 0)),
            pl.BlockSpec((QB, 2 * NS), lambda i: (i, 0)),
        ],
        out_shape=[
            jax.ShapeDtypeStruct((NBQ, EMBED), jnp.bfloat16),
            jax.ShapeDtypeStruct((NBQ, 2 * NS), jnp.int32),
        ],
        interpret=interpret,
    )(qf, W_val, b_val, woffx, boffx, woffy, boffy, W_attn, b_attn)


def _splat(vec, lane):
    return lax.gather(
        vec, jnp.zeros((16, 1), jnp.int32) + lane,
        lax.GatherDimensionNumbers(offset_dims=(), collapsed_slice_dims=(0,),
                                   start_index_map=(0,)),
        (1,), mode=lax.GatherScatterMode.PROMISE_IN_BOUNDS)


def _sc_body(table_hbm, pk_hbm, out_hbm, pkA, pkB, rowsA, rowsB, outA, outB,
             semIOA, semIOB, semGA, semGB, semOA, semOB):
    wid = lax.axis_index("s") * SC_CORES + lax.axis_index("c")
    q_base = wid * QPW

    def io_copy(ch, pk_v, sem):
        return pltpu.make_async_copy(
            pk_hbm.at[pl.ds(q_base + ch * TQ, TQ)], pk_v, sem)

    def g_copies(pk_v, rows_v, sem):
        return [
            pltpu.make_async_copy(table_hbm.at[pk_v.at[k, pl.ds(0, NS)]],
                                  rows_v.at[pl.ds(k * NS, NS)], sem)
            for k in range(TQ)
        ]

    def o_copy(ch, out_v, sem):
        return pltpu.make_async_copy(
            out_v, out_hbm.at[pl.ds((q_base + ch * TQ) * HEADS, TQ * HEADS)],
            sem)

    def compute(pk_v, rows_v, out_v):
        def q_body(qq, carry):
            base = qq * NS
            for hg in range(2):
                w16 = [
                    plsc.bitcast(
                        pk_v[qq, pl.ds(NS + c4 * 32 + hg * 16, 16)],
                        jnp.float32)
                    for c4 in range(4)
                ]
                for h4 in range(4):
                    accE = jnp.zeros((16,), jnp.float32)
                    accO = jnp.zeros((16,), jnp.float32)
                    for c4 in range(4):
                        for p in range(POINTS):
                            lane = h4 * 4 + p
                            spl = _splat(w16[c4], lane)
                            pos = base + c4 * 32 + hg * 16 + lane
                            ev, od = plsc.unpack(
                                rows_v[pos, :],
                                format=plsc.PackFormat.INTERLEAVED)
                            accE = accE + spl * ev
                            accO = accO + spl * od
                    h = hg * 4 + h4
                    out_v[qq * HEADS + h, pl.ds(0, 16)] = accE
                    out_v[qq * HEADS + h, pl.ds(16, 16)] = accO
            return carry

        lax.fori_loop(0, TQ, q_body, 0)

    c0 = io_copy(0, pkA, semIOA)
    c0.start()
    c0.wait()
    for c in g_copies(pkA, rowsA, semGA):
        c.start()
    io_copy(1, pkB, semIOB).start()

    def pair_body(i, carry):
        k0 = 2 * i

        def half(ch, pk_v, rows_v, out_v, semIO, semG, semO,
                 pk_o, rows_o, semIO_o, semG_o):
            @pl.when(ch + 1 <= NCH - 1)
            def _():
                io_copy(ch + 1, pk_o, semIO_o).wait()
                for c in g_copies(pk_o, rows_o, semG_o):
                    c.start()

            for c in g_copies(pk_v, rows_v, semG):
                c.wait()

            @pl.when(ch >= 2)
            def _():
                o_copy(ch - 2, out_v, semO).wait()

            compute(pk_v, rows_v, out_v)
            o_copy(ch, out_v, semO).start()

            @pl.when(ch + 2 <= NCH - 1)
            def _():
                io_copy(ch + 2, pk_v, semIO).start()

        half(k0, pkA, rowsA, outA, semIOA, semGA, semOA,
             pkB, rowsB, semIOB, semGB)
        half(k0 + 1, pkB, rowsB, outB, semIOB, semGB, semOB,
             pkA, rowsA, semIOA, semGA)
        return carry

    lax.fori_loop(0, NCH // 2, pair_body, 0)

    o_copy(NCH - 2, outA, semOA).wait()
    o_copy(NCH - 1, outB, semOB).wait()


def _sc_call(table, pk):
    mesh = plsc.VectorSubcoreMesh(core_axis_name="c", subcore_axis_name="s")
    return pl.kernel(
        _sc_body,
        out_type=jax.ShapeDtypeStruct((NROWS, HEAD_DIM), jnp.float32),
        mesh=mesh,
        scratch_types=[
            pltpu.VMEM((TQ, 2 * NS), jnp.int32),
            pltpu.VMEM((TQ, 2 * NS), jnp.int32),
            pltpu.VMEM((SAMP, HEAD_DIM), jnp.bfloat16),
            pltpu.VMEM((SAMP, HEAD_DIM), jnp.bfloat16),
            pltpu.VMEM((TQ * HEADS, HEAD_DIM), jnp.float32),
            pltpu.VMEM((TQ * HEADS, HEAD_DIM), jnp.float32),
            pltpu.SemaphoreType.DMA,
            pltpu.SemaphoreType.DMA,
            pltpu.SemaphoreType.DMA,
            pltpu.SemaphoreType.DMA,
            pltpu.SemaphoreType.DMA,
            pltpu.SemaphoreType.DMA,
        ],
        compiler_params=pltpu.CompilerParams(needs_layout_passes=False,
                                             use_tc_tiling_on_sc=False),
    )(table, pk)


def _out_body(s_ref, q_ref, wout_ref, bout_ref, o_ref):
    o_ref[:] = (jnp.dot(s_ref[:], wout_ref[:], preferred_element_type=jnp.float32)
                + bout_ref[:] + 2.0 * q_ref[:])


def _out_call(smp, qf, W_out, b_out, interpret=False):
    return pl.pallas_call(
        _out_body,
        grid=(GRID,),
        in_specs=[
            pl.BlockSpec((QB, EMBED), lambda i: (i, 0)),
            pl.BlockSpec((QB, EMBED), lambda i: (i, 0)),
            pl.BlockSpec((EMBED, EMBED), lambda i: (0, 0)),
            pl.BlockSpec((1, EMBED), lambda i: (0, 0)),
        ],
        out_specs=pl.BlockSpec((QB, EMBED), lambda i: (i, 0)),
        out_shape=jax.ShapeDtypeStruct((NBQ, EMBED), jnp.float32),
        interpret=interpret,
    )(smp, qf, W_out, b_out)


_UNPACK_ORDER = np.concatenate([np.arange(0, 32, 2), np.arange(1, 32, 2)])
_WOUT_PERM = np.concatenate([h * 32 + _UNPACK_ORDER for h in range(HEADS)])


def kernel(query, W_off, b_off, W_attn, b_attn, W_val, b_val, W_out, b_out):
    qf = query.reshape(NBQ, EMBED)
    woffx = W_off[:, 0::2]
    woffy = W_off[:, 1::2]
    boffx = b_off[0::2].reshape(1, 32)
    boffy = b_off[1::2].reshape(1, 32)

    val, pk = _prep_call(qf, W_val, b_val.reshape(1, EMBED), woffx, boffx,
                         woffy, boffy, W_attn, b_attn.reshape(1, 32))
    table = val.reshape(NROWS, HEAD_DIM)
    smp = _sc_call(table, pk)
    out = _out_call(smp.reshape(NBQ, EMBED), qf, W_out[_WOUT_PERM, :],
                    b_out.reshape(1, EMBED))
    return out.reshape(BS, NQ, EMBED)
